# baseline (speedup 1.0000x reference)
import functools

import jax
import jax.numpy as jnp
import numpy as np
from jax.experimental import pallas as pl
from jax.experimental.pallas import tpu as pltpu
from jax.experimental.pallas import tpu_sc as plsc

DIM = 384
NUM_HEADS = 8
HD = DIM // NUM_HEADS
WS = 7
BS = 4
STRIDE = 2
NSEL = 16
B = 8
H = 32
W = 32
N = H * W
NBLK = 225
NBLK2 = 64
DIN = DIM * BS * BS
SCALE = HD ** -0.5

PREC = jax.lax.Precision.DEFAULT


def _erf(x):
    ax = jnp.abs(x)
    t = 1.0 / (1.0 + 0.3275911 * ax)
    poly = t * (0.254829592 + t * (-0.284496736 + t * (1.421413741
               + t * (-1.453152027 + t * 1.061405429))))
    y = 1.0 - poly * jnp.exp(-ax * ax)
    return jnp.where(x < 0, -y, y)


def _gelu(x):
    return 0.5 * x * (1.0 + _erf(x * (2.0 ** -0.5)))


def _dot(a, b, prec=PREC):
    return jax.lax.dot_general(a, b, (((1,), (0,)), ((), ())),
                               precision=prec,
                               preferred_element_type=jnp.float32)


def _dot_t(a, b, prec=PREC):
    return jax.lax.dot_general(a, b, (((1,), (1,)), ((), ())),
                               precision=prec,
                               preferred_element_type=jnp.float32)


def _matmul_bias(x, w, b, tile_m):
    M, K = x.shape
    P = w.shape[1]
    assert M % tile_m == 0

    def body(x_ref, w_ref, b_ref, o_ref):
        o_ref[...] = _dot(x_ref[...], w_ref[...]) + b_ref[...]

    return pl.pallas_call(
        body,
        grid=(M // tile_m,),
        in_specs=[
            pl.BlockSpec((tile_m, K), lambda i: (i, 0)),
            pl.BlockSpec((K, P), lambda i: (0, 0)),
            pl.BlockSpec((1, P), lambda i: (0, 0)),
        ],
        out_specs=pl.BlockSpec((tile_m, P), lambda i: (i, 0)),
        out_shape=jax.ShapeDtypeStruct((M, P), jnp.float32),
    )(x, w, b.reshape(1, P))


_NB = (H - BS) // STRIDE + 1


def _compress_half(planes, padd, w1r, b1, w2, b2):
    def body(x_ref, p_ref, w1_ref, b1_ref, w2_ref, b2_ref, o_ref):
        xx = x_ref[0]
        pieces = []
        for p in range(BS * BS):
            di, dj = p // BS, p % BS
            q = (di % 2) * 2 + (dj % 2)
            a, bb = di // 2, dj // 2
            plane = xx[q * 256:(q + 1) * 256].reshape(16, 16, DIM)
            sl = plane[a:a + _NB, bb:bb + _NB].reshape(NBLK, DIM)
            pieces.append(sl + p_ref[p][None, :])
        xin = jnp.concatenate(pieces, axis=1)
        h = _gelu(_dot(xin, w1_ref[...]) + b1_ref[...])
        o_ref[0] = _dot(h, w2_ref[...]) + b2_ref[...]

    return pl.pallas_call(
        body,
        grid=(B,),
        in_specs=[
            pl.BlockSpec((1, 1024, DIM), lambda b: (b, 0, 0)),
            pl.BlockSpec((BS * BS, DIM), lambda b: (0, 0)),
            pl.BlockSpec((DIN, 2 * DIM), lambda b: (0, 0)),
            pl.BlockSpec((1, 2 * DIM), lambda b: (0, 0)),
            pl.BlockSpec((2 * DIM, DIM), lambda b: (0, 0)),
            pl.BlockSpec((1, DIM), lambda b: (0, 0)),
        ],
        out_specs=pl.BlockSpec((1, NBLK, DIM), lambda b: (b, 0, 0)),
        out_shape=jax.ShapeDtypeStruct((B, NBLK, DIM), jnp.float32),
    )(planes, padd, w1r, b1.reshape(1, -1), w2, b2.reshape(1, -1))


def _deinterleave(x4):
    pls = []
    for pr in range(2):
        for pc in range(2):
            pls.append(x4[:, pr::2, pc::2, :].reshape(B, 256, DIM))
    return jnp.concatenate(pls, axis=1)


def _attention(q, k, v, want_importance):
    L = k.shape[1]

    def body(q_ref, k_ref, v_ref, o_ref, *maybe_imp):
        qq = q_ref[0]
        kk = k_ref[0]
        vv = v_ref[0]
        outs = []
        imp = jnp.zeros((1, L), jnp.float32)
        for h in range(NUM_HEADS):
            sl = slice(h * HD, (h + 1) * HD)
            s = _dot_t(qq[:, sl], kk[:, sl]) * SCALE
            m = jnp.max(s, axis=-1, keepdims=True)
            e = jnp.exp(s - m)
            p = e * (1.0 / jnp.sum(e, axis=-1, keepdims=True))
            outs.append(_dot(p, vv[:, sl]))
            if want_importance:
                imp = imp + jnp.sum(p, axis=0, keepdims=True)
        o_ref[0] = jnp.concatenate(outs, axis=-1)
        if want_importance:
            maybe_imp[0][0] = imp

    out_shapes = [jax.ShapeDtypeStruct((B, N, DIM), jnp.float32)]
    out_specs = [pl.BlockSpec((1, N, DIM), lambda b: (b, 0, 0))]
    if want_importance:
        out_shapes.append(jax.ShapeDtypeStruct((B, 1, L), jnp.float32))
        out_specs.append(pl.BlockSpec((1, 1, L), lambda b: (b, 0, 0)))

    res = pl.pallas_call(
        body,
        grid=(B,),
        in_specs=[
            pl.BlockSpec((1, N, DIM), lambda b: (b, 0, 0)),
            pl.BlockSpec((1, L, DIM), lambda b: (b, 0, 0)),
            pl.BlockSpec((1, L, DIM), lambda b: (b, 0, 0)),
        ],
        out_specs=out_specs,
        out_shape=out_shapes,
    )(q, k, v)
    return res if want_importance else res[0]


_NB2 = (H - BS) // BS + 1


def _select_indices(imp):
    def body(imp_ref, o_ref):
        val = imp_ref[0]
        lanes = jax.lax.broadcasted_iota(jnp.int32, (1, NBLK), 1)
        lanes128 = jax.lax.broadcasted_iota(jnp.int32, (1, 128), 1)
        acc = jnp.zeros((1, 128), jnp.int32)
        for s in range(NSEL):
            idx = jnp.argmax(val, axis=-1)[0].astype(jnp.int32)
            val = jnp.where(lanes == idx, -jnp.inf, val)
            cidx = jnp.minimum(idx, NBLK2 - 1)
            acc = jnp.where(lanes128 == s, cidx, acc)
        o_ref[0] = acc

    return pl.pallas_call(
        body,
        grid=(B,),
        in_specs=[pl.BlockSpec((1, 1, NBLK), lambda b: (b, 0, 0))],
        out_specs=pl.BlockSpec((1, 1, 128), lambda b: (b, 0, 0)),
        out_shape=jax.ShapeDtypeStruct((B, 1, 128), jnp.int32),
    )(imp)


def _sc_gather(idxf, kblkf, vblkf):
    blk_elems = BS * BS * DIM
    half = B // 2
    out_t = [jax.ShapeDtypeStruct((B * NSEL * blk_elems,), jnp.float32)] * 2
    mesh = plsc.ScalarSubcoreMesh(axis_name="core", num_cores=2)

    @pl.kernel(out_type=out_t, mesh=mesh,
               scratch_types=[pltpu.SMEM((half * 128,), jnp.int32),
                              pltpu.SemaphoreType.DMA,
                              pltpu.SemaphoreType.DMA])
    def sc_kernel(idx_ref, k_ref, v_ref, ko_ref, vo_ref, sidx, sem_in,
                  sem_out):
        core = jax.lax.axis_index("core")
        pltpu.async_copy(idx_ref.at[pl.ds(core * (half * 128), half * 128)],
                         sidx, sem_in).wait()
        copies = []
        for bloc in range(half):
            b = core * half + bloc
            for s in range(NSEL):
                cidx = sidx[bloc * 128 + s]
                src = (b * NBLK2 + cidx) * blk_elems
                dst = (b * NSEL + s) * blk_elems
                copies.append(pltpu.async_copy(
                    k_ref.at[pl.ds(src, blk_elems)],
                    ko_ref.at[pl.ds(dst, blk_elems)], sem_out))
                copies.append(pltpu.async_copy(
                    v_ref.at[pl.ds(src, blk_elems)],
                    vo_ref.at[pl.ds(dst, blk_elems)], sem_out))
        for c in copies:
            c.wait()

    return sc_kernel(idxf, kblkf, vblkf)


def _relayout_blocks(raw):
    def body(r_ref, o_ref):
        for s in range(NSEL):
            g = r_ref[0][s * BS * BS:(s + 1) * BS * BS]
            o_ref[0, pl.ds(s * DIM, DIM), :] = g.T

    return pl.pallas_call(
        body,
        grid=(B,),
        in_specs=[pl.BlockSpec((1, NSEL * BS * BS, DIM), lambda b: (b, 0, 0))],
        out_specs=pl.BlockSpec((1, NSEL * DIM, BS * BS), lambda b: (b, 0, 0)),
        out_shape=jax.ShapeDtypeStruct((B, NSEL * DIM, BS * BS), jnp.float32),
    )(raw)


def _blockify(x4):
    y = x4.reshape(B, _NB2, BS, _NB2, BS, DIM).transpose(0, 1, 3, 2, 4, 5)
    return y.reshape(B, NBLK2 * BS * BS, DIM)


_NW = 5
_WT = WS * WS
_WINB = 8
_NWIN = B * _NW * _NW


def _rel_pos_index_np(ws):
    coords = np.stack(np.meshgrid(np.arange(ws), np.arange(ws), indexing="ij"))
    cf = coords.reshape(2, -1)
    rel = (cf[:, :, None] - cf[:, None, :]).transpose(1, 2, 0).astype(np.int64).copy()
    rel[:, :, 0] += ws - 1
    rel[:, :, 1] += ws - 1
    rel[:, :, 0] *= 2 * ws - 1
    return rel.sum(-1)


_RPI = _rel_pos_index_np(WS)


_GT = _WINB * _WT


def _window_attn(xg, wqkv, bqkv, wpr, bpr, bias_big):
    def body(x_ref, wqkv_ref, bqkv_ref, bias_ref, wpr_ref, bpr_ref, o_ref):
        xx = x_ref[0]
        qkv = _dot(xx, wqkv_ref[...]) + bqkv_ref[...]
        outs = []
        for h in range(NUM_HEADS):
            qh = qkv[:, h * HD:(h + 1) * HD]
            kh = qkv[:, DIM + h * HD:DIM + (h + 1) * HD]
            vh = qkv[:, 2 * DIM + h * HD:2 * DIM + (h + 1) * HD]
            s = _dot_t(qh, kh) * SCALE + bias_ref[h]
            m = jnp.max(s, axis=-1, keepdims=True)
            e = jnp.exp(s - m)
            p = e * (1.0 / jnp.sum(e, axis=-1, keepdims=True))
            outs.append(_dot(p, vh))
        o = jnp.concatenate(outs, axis=-1)
        o_ref[0] = _dot(o, wpr_ref[...]) + bpr_ref[...]

    return pl.pallas_call(
        body,
        grid=(_NWIN // _WINB,),
        in_specs=[
            pl.BlockSpec((1, _GT, DIM), lambda i: (i, 0, 0)),
            pl.BlockSpec((DIM, 3 * DIM), lambda i: (0, 0)),
            pl.BlockSpec((1, 3 * DIM), lambda i: (0, 0)),
            pl.BlockSpec((NUM_HEADS, _GT, _GT), lambda i: (0, 0, 0)),
            pl.BlockSpec((DIM, DIM), lambda i: (0, 0)),
            pl.BlockSpec((1, DIM), lambda i: (0, 0)),
        ],
        out_specs=pl.BlockSpec((1, _GT, DIM), lambda i: (i, 0, 0)),
        out_shape=jax.ShapeDtypeStruct((_NWIN // _WINB, _GT, DIM),
                                       jnp.float32),
    )(xg, wqkv, bqkv.reshape(1, -1), bias_big, wpr, bpr.reshape(1, -1))


def _combine(x_seq, o_cmp, o_slc, o_win, wg, bg, wp, bp):
    M = x_seq.shape[0]
    TM = 1024

    def body(x_ref, c_ref, s_ref, w_ref, wg_ref, bg_ref, wp_ref, bp_ref, o_ref):
        g = jax.nn.sigmoid(_dot(x_ref[...], wg_ref[...]) + bg_ref[...])
        comb = (g[:, 0:1] * c_ref[...] + g[:, 1:2] * s_ref[...]
                + g[:, 2:3] * w_ref[...])
        o_ref[...] = _dot(comb, wp_ref[...]) + bp_ref[...]

    return pl.pallas_call(
        body,
        grid=(M // TM,),
        in_specs=[
            pl.BlockSpec((TM, DIM), lambda i: (i, 0)),
            pl.BlockSpec((TM, DIM), lambda i: (i, 0)),
            pl.BlockSpec((TM, DIM), lambda i: (i, 0)),
            pl.BlockSpec((TM, DIM), lambda i: (i, 0)),
            pl.BlockSpec((DIM, 3), lambda i: (0, 0)),
            pl.BlockSpec((1, 3), lambda i: (0, 0)),
            pl.BlockSpec((DIM, DIM), lambda i: (0, 0)),
            pl.BlockSpec((1, DIM), lambda i: (0, 0)),
        ],
        out_specs=pl.BlockSpec((TM, DIM), lambda i: (i, 0)),
        out_shape=jax.ShapeDtypeStruct((M, DIM), jnp.float32),
    )(x_seq, o_cmp, o_slc, o_win, wg, bg.reshape(1, 3), wp, bp.reshape(1, DIM))


def kernel(x, W_qkv, b_qkv, pos_embed, Wk1, bk1, Wk2, bk2, Wv1, bv1, Wv2, bv2,
           W_qkv_w, b_qkv_w, W_proj_w, b_proj_w, rel_bias, W_gate, b_gate,
           W_proj, b_proj):
    x_seq = x.reshape(B, DIM, N).transpose(0, 2, 1)
    xf = x_seq.reshape(B * N, DIM)

    qkv = _matmul_bias(xf, W_qkv, b_qkv, 1024).reshape(B, N, 3 * DIM)
    q = qkv[:, :, :DIM]
    k2 = qkv[:, :, DIM:2 * DIM]
    v2 = qkv[:, :, 2 * DIM:]
    k4 = k2.reshape(B, H, W, DIM)
    v4 = v2.reshape(B, H, W, DIM)

    w1kr = Wk1.reshape(DIM, BS * BS, 2 * DIM).transpose(1, 0, 2).reshape(DIN, 2 * DIM)
    w1vr = Wv1.reshape(DIM, BS * BS, 2 * DIM).transpose(1, 0, 2).reshape(DIN, 2 * DIM)
    padd = pos_embed.reshape(DIN).reshape(DIM, BS * BS).T
    kcmp = _compress_half(_deinterleave(k4), padd, w1kr, bk1, Wk2, bk2)
    vcmp = _compress_half(_deinterleave(v4), jnp.zeros_like(padd), w1vr, bv1,
                          Wv2, bv2)

    out_cmp, imp = _attention(q, kcmp, vcmp, True)

    idxf = _select_indices(imp).reshape(B * 128)
    kblkf = _blockify(k4).reshape(B * NBLK2 * BS * BS * DIM)
    vblkf = _blockify(v4).reshape(B * NBLK2 * BS * BS * DIM)
    kraw, vraw = _sc_gather(idxf, kblkf, vblkf)
    kslc = _relayout_blocks(kraw.reshape(B, NSEL * BS * BS, DIM)).reshape(
        B, NSEL * BS * BS, DIM)
    vslc = _relayout_blocks(vraw.reshape(B, NSEL * BS * BS, DIM)).reshape(
        B, NSEL * BS * BS, DIM)

    out_slc = _attention(q, kslc, vslc, False)

    xp = jnp.pad(x, ((0, 0), (0, 0), (0, _NW * WS - H), (0, _NW * WS - W)))
    xw = xp.reshape(B, DIM, _NW, WS, _NW, WS).transpose(0, 2, 4, 3, 5, 1)
    xg = xw.reshape(_NWIN // _WINB, _GT, DIM)
    bias = rel_bias[jnp.asarray(_RPI.reshape(-1))].reshape(_WT, _WT, NUM_HEADS)
    bias = bias.transpose(2, 0, 1)
    blockmask = jnp.kron(jnp.eye(_WINB, dtype=jnp.float32),
                         jnp.ones((_WT, _WT), jnp.float32)) > 0.5
    bias_big = jnp.where(blockmask[None], jnp.tile(bias, (1, _WINB, _WINB)),
                         -1e30)
    ow = _window_attn(xg, W_qkv_w, b_qkv_w, W_proj_w, b_proj_w, bias_big)
    ow = ow.reshape(B, _NW, _NW, WS, WS, DIM).transpose(0, 5, 1, 3, 2, 4)
    ow = ow.reshape(B, DIM, _NW * WS, _NW * WS)[:, :, :H, :W]
    out_win = ow.reshape(B, DIM, N).transpose(0, 2, 1).reshape(B * N, DIM)

    out = _combine(xf, out_cmp.reshape(B * N, DIM), out_slc.reshape(B * N, DIM),
                   out_win, W_gate, b_gate, W_proj, b_proj)
    return out.reshape(B, N, DIM).transpose(0, 2, 1).reshape(B, DIM, H, W)


# speedup vs baseline: 1.1435x; 1.1435x over previous
import functools

import jax
import jax.numpy as jnp
import numpy as np
from jax.experimental import pallas as pl
from jax.experimental.pallas import tpu as pltpu
from jax.experimental.pallas import tpu_sc as plsc

DIM = 384
NUM_HEADS = 8
HD = DIM // NUM_HEADS
WS = 7
BS = 4
STRIDE = 2
NSEL = 16
B = 8
H = 32
W = 32
N = H * W
NBLK = 225
NBLK2 = 64
DIN = DIM * BS * BS
SCALE = HD ** -0.5

PREC = jax.lax.Precision.DEFAULT


def _erf(x):
    ax = jnp.abs(x)
    t = 1.0 / (1.0 + 0.3275911 * ax)
    poly = t * (0.254829592 + t * (-0.284496736 + t * (1.421413741
               + t * (-1.453152027 + t * 1.061405429))))
    y = 1.0 - poly * jnp.exp(-ax * ax)
    return jnp.where(x < 0, -y, y)


def _gelu(x):
    return 0.5 * x * (1.0 + _erf(x * (2.0 ** -0.5)))


def _dot(a, b, prec=PREC):
    return jax.lax.dot_general(a, b, (((1,), (0,)), ((), ())),
                               precision=prec,
                               preferred_element_type=jnp.float32)


def _dot_t(a, b, prec=PREC):
    return jax.lax.dot_general(a, b, (((1,), (1,)), ((), ())),
                               precision=prec,
                               preferred_element_type=jnp.float32)


def _matmul_bias(x, w, b, tile_m):
    M, K = x.shape
    P = w.shape[1]
    assert M % tile_m == 0

    def body(x_ref, w_ref, b_ref, o_ref):
        o_ref[...] = _dot(x_ref[...], w_ref[...]) + b_ref[...]

    return pl.pallas_call(
        body,
        grid=(M // tile_m,),
        in_specs=[
            pl.BlockSpec((tile_m, K), lambda i: (i, 0)),
            pl.BlockSpec((K, P), lambda i: (0, 0)),
            pl.BlockSpec((1, P), lambda i: (0, 0)),
        ],
        out_specs=pl.BlockSpec((tile_m, P), lambda i: (i, 0)),
        out_shape=jax.ShapeDtypeStruct((M, P), jnp.float32),
    )(x, w, b.reshape(1, P))


_NB = (H - BS) // STRIDE + 1


def _compress_half(planes, padd, w1r, b1, w2, b2):
    def body(x_ref, p_ref, w1_ref, b1_ref, w2_ref, b2_ref, o_ref):
        xx = x_ref[0]
        pieces = []
        for p in range(BS * BS):
            di, dj = p // BS, p % BS
            q = (di % 2) * 2 + (dj % 2)
            a, bb = di // 2, dj // 2
            plane = xx[q * 256:(q + 1) * 256].reshape(16, 16, DIM)
            sl = plane[a:a + _NB, bb:bb + _NB].reshape(NBLK, DIM)
            pieces.append(sl + p_ref[p][None, :])
        xin = jnp.concatenate(pieces, axis=1)
        h = _gelu(_dot(xin, w1_ref[...]) + b1_ref[...])
        o_ref[0] = _dot(h, w2_ref[...]) + b2_ref[...]

    return pl.pallas_call(
        body,
        grid=(B,),
        in_specs=[
            pl.BlockSpec((1, 1024, DIM), lambda b: (b, 0, 0)),
            pl.BlockSpec((BS * BS, DIM), lambda b: (0, 0)),
            pl.BlockSpec((DIN, 2 * DIM), lambda b: (0, 0)),
            pl.BlockSpec((1, 2 * DIM), lambda b: (0, 0)),
            pl.BlockSpec((2 * DIM, DIM), lambda b: (0, 0)),
            pl.BlockSpec((1, DIM), lambda b: (0, 0)),
        ],
        out_specs=pl.BlockSpec((1, NBLK, DIM), lambda b: (b, 0, 0)),
        out_shape=jax.ShapeDtypeStruct((B, NBLK, DIM), jnp.float32),
    )(planes, padd, w1r, b1.reshape(1, -1), w2, b2.reshape(1, -1))


def _deinterleave(x4):
    pls = []
    for pr in range(2):
        for pc in range(2):
            pls.append(x4[:, pr::2, pc::2, :].reshape(B, 256, DIM))
    return jnp.concatenate(pls, axis=1)


def _attention(q, k, v, want_importance):
    L = k.shape[1]

    def body(q_ref, k_ref, v_ref, o_ref, *maybe_imp):
        qq = q_ref[0]
        kk = k_ref[0]
        vv = v_ref[0]
        outs = []
        imp = jnp.zeros((1, L), jnp.float32)
        for h in range(NUM_HEADS):
            sl = slice(h * HD, (h + 1) * HD)
            s = _dot_t(qq[:, sl], kk[:, sl]) * SCALE
            m = jnp.max(s, axis=-1, keepdims=True)
            e = jnp.exp(s - m)
            p = e * (1.0 / jnp.sum(e, axis=-1, keepdims=True))
            outs.append(_dot(p, vv[:, sl]))
            if want_importance:
                imp = imp + jnp.sum(p, axis=0, keepdims=True)
        o_ref[0] = jnp.concatenate(outs, axis=-1)
        if want_importance:
            maybe_imp[0][0] = imp

    out_shapes = [jax.ShapeDtypeStruct((B, N, DIM), jnp.float32)]
    out_specs = [pl.BlockSpec((1, N, DIM), lambda b: (b, 0, 0))]
    if want_importance:
        out_shapes.append(jax.ShapeDtypeStruct((B, 1, L), jnp.float32))
        out_specs.append(pl.BlockSpec((1, 1, L), lambda b: (b, 0, 0)))

    res = pl.pallas_call(
        body,
        grid=(B,),
        in_specs=[
            pl.BlockSpec((1, N, DIM), lambda b: (b, 0, 0)),
            pl.BlockSpec((1, L, DIM), lambda b: (b, 0, 0)),
            pl.BlockSpec((1, L, DIM), lambda b: (b, 0, 0)),
        ],
        out_specs=out_specs,
        out_shape=out_shapes,
    )(q, k, v)
    return res if want_importance else res[0]


_NB2 = (H - BS) // BS + 1


_GROWS = B * NSEL * BS * BS
_NWORK = 32


def _select_rows(imp):
    def body(imp_ref, o_ref):
        b = pl.program_id(0)
        val = imp_ref[0]
        lanes = jax.lax.broadcasted_iota(jnp.int32, (1, NBLK), 1)
        g = (jax.lax.broadcasted_iota(jnp.int32, (8, 128), 0) * 128
             + jax.lax.broadcasted_iota(jnp.int32, (8, 128), 1))
        s_of_g = g // (BS * BS)
        acc = jnp.zeros((8, 128), jnp.int32)
        for s in range(NSEL):
            idx = jnp.argmax(val, axis=-1)[0].astype(jnp.int32)
            val = jnp.where(lanes == idx, -jnp.inf, val)
            cidx = jnp.minimum(idx, NBLK2 - 1)
            acc = jnp.where(s_of_g == s, (b * NBLK2 + cidx) * (BS * BS), acc)
        o_ref[0] = acc + jnp.where(g < NSEL * BS * BS, g % (BS * BS), 0)

    return pl.pallas_call(
        body,
        grid=(B,),
        in_specs=[pl.BlockSpec((1, 1, NBLK), lambda b: (b, 0, 0))],
        out_specs=pl.BlockSpec((1, 8, 128), lambda b: (b, 0, 0)),
        out_shape=jax.ShapeDtypeStruct((B, 8, 128), jnp.int32),
    )(imp)


def _sc_gather(idxf, kblk2, vblk2):
    per_w = _GROWS // _NWORK
    out_t = [jax.ShapeDtypeStruct((_GROWS, DIM), jnp.float32)] * 2
    mesh = plsc.VectorSubcoreMesh(core_axis_name="c", subcore_axis_name="s")

    @pl.kernel(out_type=out_t, mesh=mesh,
               scratch_types=[pltpu.VMEM((per_w,), jnp.int32),
                              pltpu.VMEM((per_w, DIM), jnp.float32),
                              pltpu.VMEM((per_w, DIM), jnp.float32),
                              pltpu.SemaphoreType.DMA,
                              pltpu.SemaphoreType.DMA])
    def sc_kernel(idx_ref, k_ref, v_ref, ko_ref, vo_ref, idx_v, rk, rv,
                  semk, semv):
        wid = jax.lax.axis_index("s") * 2 + jax.lax.axis_index("c")
        base = wid * per_w
        pltpu.sync_copy(idx_ref.at[pl.ds(base, per_w)], idx_v)
        ck = pltpu.async_copy(k_ref.at[idx_v], rk, semk)
        cv = pltpu.async_copy(v_ref.at[idx_v], rv, semv)
        ck.wait()
        pltpu.sync_copy(rk, ko_ref.at[pl.ds(base, per_w)])
        cv.wait()
        pltpu.sync_copy(rv, vo_ref.at[pl.ds(base, per_w)])

    return sc_kernel(idxf, kblk2, vblk2)


def _relayout_blocks(raw):
    def body(r_ref, o_ref):
        for s in range(NSEL):
            g = r_ref[0][s * BS * BS:(s + 1) * BS * BS]
            o_ref[0, pl.ds(s * DIM, DIM), :] = g.T

    return pl.pallas_call(
        body,
        grid=(B,),
        in_specs=[pl.BlockSpec((1, NSEL * BS * BS, DIM), lambda b: (b, 0, 0))],
        out_specs=pl.BlockSpec((1, NSEL * DIM, BS * BS), lambda b: (b, 0, 0)),
        out_shape=jax.ShapeDtypeStruct((B, NSEL * DIM, BS * BS), jnp.float32),
    )(raw)


def _blockify(x4):
    y = x4.reshape(B, _NB2, BS, _NB2, BS, DIM).transpose(0, 1, 3, 2, 4, 5)
    return y.reshape(B, NBLK2 * BS * BS, DIM)


_NW = 5
_WT = WS * WS
_WINB = 8
_NWIN = B * _NW * _NW


def _rel_pos_index_np(ws):
    coords = np.stack(np.meshgrid(np.arange(ws), np.arange(ws), indexing="ij"))
    cf = coords.reshape(2, -1)
    rel = (cf[:, :, None] - cf[:, None, :]).transpose(1, 2, 0).astype(np.int64).copy()
    rel[:, :, 0] += ws - 1
    rel[:, :, 1] += ws - 1
    rel[:, :, 0] *= 2 * ws - 1
    return rel.sum(-1)


_RPI = _rel_pos_index_np(WS)


_GT = _WINB * _WT


def _window_attn(xg, wqkv, bqkv, wpr, bpr, bias_big):
    def body(x_ref, wqkv_ref, bqkv_ref, bias_ref, wpr_ref, bpr_ref, o_ref):
        xx = x_ref[0]
        qkv = _dot(xx, wqkv_ref[...]) + bqkv_ref[...]
        outs = []
        for h in range(NUM_HEADS):
            qh = qkv[:, h * HD:(h + 1) * HD]
            kh = qkv[:, DIM + h * HD:DIM + (h + 1) * HD]
            vh = qkv[:, 2 * DIM + h * HD:2 * DIM + (h + 1) * HD]
            s = _dot_t(qh, kh) * SCALE + bias_ref[h]
            m = jnp.max(s, axis=-1, keepdims=True)
            e = jnp.exp(s - m)
            p = e * (1.0 / jnp.sum(e, axis=-1, keepdims=True))
            outs.append(_dot(p, vh))
        o = jnp.concatenate(outs, axis=-1)
        o_ref[0] = _dot(o, wpr_ref[...]) + bpr_ref[...]

    return pl.pallas_call(
        body,
        grid=(_NWIN // _WINB,),
        in_specs=[
            pl.BlockSpec((1, _GT, DIM), lambda i: (i, 0, 0)),
            pl.BlockSpec((DIM, 3 * DIM), lambda i: (0, 0)),
            pl.BlockSpec((1, 3 * DIM), lambda i: (0, 0)),
            pl.BlockSpec((NUM_HEADS, _GT, _GT), lambda i: (0, 0, 0)),
            pl.BlockSpec((DIM, DIM), lambda i: (0, 0)),
            pl.BlockSpec((1, DIM), lambda i: (0, 0)),
        ],
        out_specs=pl.BlockSpec((1, _GT, DIM), lambda i: (i, 0, 0)),
        out_shape=jax.ShapeDtypeStruct((_NWIN // _WINB, _GT, DIM),
                                       jnp.float32),
    )(xg, wqkv, bqkv.reshape(1, -1), bias_big, wpr, bpr.reshape(1, -1))


def _combine(x_seq, o_cmp, o_slc, o_win, wg, bg, wp, bp):
    M = x_seq.shape[0]
    TM = 1024

    def body(x_ref, c_ref, s_ref, w_ref, wg_ref, bg_ref, wp_ref, bp_ref, o_ref):
        g = jax.nn.sigmoid(_dot(x_ref[...], wg_ref[...]) + bg_ref[...])
        comb = (g[:, 0:1] * c_ref[...] + g[:, 1:2] * s_ref[...]
                + g[:, 2:3] * w_ref[...])
        o_ref[...] = _dot(comb, wp_ref[...]) + bp_ref[...]

    return pl.pallas_call(
        body,
        grid=(M // TM,),
        in_specs=[
            pl.BlockSpec((TM, DIM), lambda i: (i, 0)),
            pl.BlockSpec((TM, DIM), lambda i: (i, 0)),
            pl.BlockSpec((TM, DIM), lambda i: (i, 0)),
            pl.BlockSpec((TM, DIM), lambda i: (i, 0)),
            pl.BlockSpec((DIM, 3), lambda i: (0, 0)),
            pl.BlockSpec((1, 3), lambda i: (0, 0)),
            pl.BlockSpec((DIM, DIM), lambda i: (0, 0)),
            pl.BlockSpec((1, DIM), lambda i: (0, 0)),
        ],
        out_specs=pl.BlockSpec((TM, DIM), lambda i: (i, 0)),
        out_shape=jax.ShapeDtypeStruct((M, DIM), jnp.float32),
    )(x_seq, o_cmp, o_slc, o_win, wg, bg.reshape(1, 3), wp, bp.reshape(1, DIM))


def kernel(x, W_qkv, b_qkv, pos_embed, Wk1, bk1, Wk2, bk2, Wv1, bv1, Wv2, bv2,
           W_qkv_w, b_qkv_w, W_proj_w, b_proj_w, rel_bias, W_gate, b_gate,
           W_proj, b_proj):
    x_seq = x.reshape(B, DIM, N).transpose(0, 2, 1)
    xf = x_seq.reshape(B * N, DIM)

    qkv = _matmul_bias(xf, W_qkv, b_qkv, 1024).reshape(B, N, 3 * DIM)
    q = qkv[:, :, :DIM]
    k2 = qkv[:, :, DIM:2 * DIM]
    v2 = qkv[:, :, 2 * DIM:]
    k4 = k2.reshape(B, H, W, DIM)
    v4 = v2.reshape(B, H, W, DIM)

    w1kr = Wk1.reshape(DIM, BS * BS, 2 * DIM).transpose(1, 0, 2).reshape(DIN, 2 * DIM)
    w1vr = Wv1.reshape(DIM, BS * BS, 2 * DIM).transpose(1, 0, 2).reshape(DIN, 2 * DIM)
    padd = pos_embed.reshape(DIN).reshape(DIM, BS * BS).T
    kcmp = _compress_half(_deinterleave(k4), padd, w1kr, bk1, Wk2, bk2)
    vcmp = _compress_half(_deinterleave(v4), jnp.zeros_like(padd), w1vr, bv1,
                          Wv2, bv2)

    out_cmp, imp = _attention(q, kcmp, vcmp, True)

    idxf = _select_rows(imp)[:, :2, :].reshape(_GROWS)
    kblk2 = _blockify(k4).reshape(B * NBLK2 * BS * BS, DIM)
    vblk2 = _blockify(v4).reshape(B * NBLK2 * BS * BS, DIM)
    kraw, vraw = _sc_gather(idxf, kblk2, vblk2)
    kslc = _relayout_blocks(kraw.reshape(B, NSEL * BS * BS, DIM)).reshape(
        B, NSEL * BS * BS, DIM)
    vslc = _relayout_blocks(vraw.reshape(B, NSEL * BS * BS, DIM)).reshape(
        B, NSEL * BS * BS, DIM)

    out_slc = _attention(q, kslc, vslc, False)

    xp = jnp.pad(x, ((0, 0), (0, 0), (0, _NW * WS - H), (0, _NW * WS - W)))
    xw = xp.reshape(B, DIM, _NW, WS, _NW, WS).transpose(0, 2, 4, 3, 5, 1)
    xg = xw.reshape(_NWIN // _WINB, _GT, DIM)
    bias = rel_bias[jnp.asarray(_RPI.reshape(-1))].reshape(_WT, _WT, NUM_HEADS)
    bias = bias.transpose(2, 0, 1)
    blockmask = jnp.kron(jnp.eye(_WINB, dtype=jnp.float32),
                         jnp.ones((_WT, _WT), jnp.float32)) > 0.5
    bias_big = jnp.where(blockmask[None], jnp.tile(bias, (1, _WINB, _WINB)),
                         -1e30)
    ow = _window_attn(xg, W_qkv_w, b_qkv_w, W_proj_w, b_proj_w, bias_big)
    ow = ow.reshape(B, _NW, _NW, WS, WS, DIM).transpose(0, 5, 1, 3, 2, 4)
    ow = ow.reshape(B, DIM, _NW * WS, _NW * WS)[:, :, :H, :W]
    out_win = ow.reshape(B, DIM, N).transpose(0, 2, 1).reshape(B * N, DIM)

    out = _combine(xf, out_cmp.reshape(B * N, DIM), out_slc.reshape(B * N, DIM),
                   out_win, W_gate, b_gate, W_proj, b_proj)
    return out.reshape(B, N, DIM).transpose(0, 2, 1).reshape(B, DIM, H, W)


# speedup vs baseline: 1.1615x; 1.0157x over previous
import functools

import jax
import jax.numpy as jnp
import numpy as np
from jax.experimental import pallas as pl
from jax.experimental.pallas import tpu as pltpu
from jax.experimental.pallas import tpu_sc as plsc

DIM = 384
NUM_HEADS = 8
HD = DIM // NUM_HEADS
WS = 7
BS = 4
STRIDE = 2
NSEL = 16
B = 8
H = 32
W = 32
N = H * W
NBLK = 225
NBLK2 = 64
DIN = DIM * BS * BS
SCALE = HD ** -0.5

PREC = jax.lax.Precision.DEFAULT


def _erf(x):
    ax = jnp.abs(x)
    t = 1.0 / (1.0 + 0.3275911 * ax)
    poly = t * (0.254829592 + t * (-0.284496736 + t * (1.421413741
               + t * (-1.453152027 + t * 1.061405429))))
    y = 1.0 - poly * jnp.exp(-ax * ax)
    return jnp.where(x < 0, -y, y)


def _gelu(x):
    return 0.5 * x * (1.0 + _erf(x * (2.0 ** -0.5)))


def _dot(a, b, prec=PREC):
    return jax.lax.dot_general(a, b, (((1,), (0,)), ((), ())),
                               precision=prec,
                               preferred_element_type=jnp.float32)


def _dot_t(a, b, prec=PREC):
    return jax.lax.dot_general(a, b, (((1,), (1,)), ((), ())),
                               precision=prec,
                               preferred_element_type=jnp.float32)


def _qkv_proj(x, w, b, tile_m):
    M, K = x.shape
    assert M % tile_m == 0

    def body(x_ref, w_ref, b_ref, oq_ref, ok_ref, ov_ref):
        y = _dot(x_ref[...], w_ref[...]) + b_ref[...]
        oq_ref[...] = y[:, :DIM]
        ok_ref[...] = y[:, DIM:2 * DIM]
        ov_ref[...] = y[:, 2 * DIM:]

    return pl.pallas_call(
        body,
        grid=(M // tile_m,),
        in_specs=[
            pl.BlockSpec((tile_m, K), lambda i: (i, 0)),
            pl.BlockSpec((K, 3 * DIM), lambda i: (0, 0)),
            pl.BlockSpec((1, 3 * DIM), lambda i: (0, 0)),
        ],
        out_specs=[pl.BlockSpec((tile_m, DIM), lambda i: (i, 0))] * 3,
        out_shape=[jax.ShapeDtypeStruct((M, DIM), jnp.float32)] * 3,
    )(x, w, b.reshape(1, 3 * DIM))


_NB = (H - BS) // STRIDE + 1


def _compress_half(planes, padd, w1r, b1, w2, b2):
    def body(x_ref, p_ref, w1_ref, b1_ref, w2_ref, b2_ref, o_ref):
        xx = x_ref[0]
        pieces = []
        for p in range(BS * BS):
            di, dj = p // BS, p % BS
            q = (di % 2) * 2 + (dj % 2)
            a, bb = di // 2, dj // 2
            plane = xx[q * 256:(q + 1) * 256].reshape(16, 16, DIM)
            sl = plane[a:a + _NB, bb:bb + _NB].reshape(NBLK, DIM)
            pieces.append(sl + p_ref[p][None, :])
        xin = jnp.concatenate(pieces, axis=1)
        h = _gelu(_dot(xin, w1_ref[...]) + b1_ref[...])
        o_ref[0] = _dot(h, w2_ref[...]) + b2_ref[...]

    return pl.pallas_call(
        body,
        grid=(B,),
        in_specs=[
            pl.BlockSpec((1, 1024, DIM), lambda b: (b, 0, 0)),
            pl.BlockSpec((BS * BS, DIM), lambda b: (0, 0)),
            pl.BlockSpec((DIN, 2 * DIM), lambda b: (0, 0)),
            pl.BlockSpec((1, 2 * DIM), lambda b: (0, 0)),
            pl.BlockSpec((2 * DIM, DIM), lambda b: (0, 0)),
            pl.BlockSpec((1, DIM), lambda b: (0, 0)),
        ],
        out_specs=pl.BlockSpec((1, NBLK, DIM), lambda b: (b, 0, 0)),
        out_shape=jax.ShapeDtypeStruct((B, NBLK, DIM), jnp.float32),
    )(planes, padd, w1r, b1.reshape(1, -1), w2, b2.reshape(1, -1))


def _deinterleave(x4):
    pls = []
    for pr in range(2):
        for pc in range(2):
            pls.append(x4[:, pr::2, pc::2, :].reshape(B, 256, DIM))
    return jnp.concatenate(pls, axis=1)


def _attention(q, k, v, want_importance):
    L = k.shape[1]

    def body(q_ref, k_ref, v_ref, o_ref, *maybe_imp):
        qq = q_ref[0]
        kk = k_ref[0]
        vv = v_ref[0]
        outs = []
        imp = jnp.zeros((1, L), jnp.float32)
        for h in range(NUM_HEADS):
            sl = slice(h * HD, (h + 1) * HD)
            s = _dot_t(qq[:, sl], kk[:, sl]) * SCALE
            m = jnp.max(s, axis=-1, keepdims=True)
            e = jnp.exp(s - m)
            p = e * (1.0 / jnp.sum(e, axis=-1, keepdims=True))
            outs.append(_dot(p, vv[:, sl]))
            if want_importance:
                imp = imp + jnp.sum(p, axis=0, keepdims=True)
        o_ref[0] = jnp.concatenate(outs, axis=-1)
        if want_importance:
            maybe_imp[0][0] = imp

    out_shapes = [jax.ShapeDtypeStruct((B, N, DIM), jnp.float32)]
    out_specs = [pl.BlockSpec((1, N, DIM), lambda b: (b, 0, 0))]
    if want_importance:
        out_shapes.append(jax.ShapeDtypeStruct((B, 1, L), jnp.float32))
        out_specs.append(pl.BlockSpec((1, 1, L), lambda b: (b, 0, 0)))

    res = pl.pallas_call(
        body,
        grid=(B,),
        in_specs=[
            pl.BlockSpec((1, N, DIM), lambda b: (b, 0, 0)),
            pl.BlockSpec((1, L, DIM), lambda b: (b, 0, 0)),
            pl.BlockSpec((1, L, DIM), lambda b: (b, 0, 0)),
        ],
        out_specs=out_specs,
        out_shape=out_shapes,
    )(q, k, v)
    return res if want_importance else res[0]


_NB2 = (H - BS) // BS + 1


_GROWS = B * NSEL * BS * BS
_NWORK = 32


def _select_rows(imp):
    def body(imp_ref, o_ref):
        b = pl.program_id(0)
        val = imp_ref[0]
        lanes = jax.lax.broadcasted_iota(jnp.int32, (1, NBLK), 1)
        g = (jax.lax.broadcasted_iota(jnp.int32, (8, 128), 0) * 128
             + jax.lax.broadcasted_iota(jnp.int32, (8, 128), 1))
        s_of_g = g // (BS * BS)
        acc = jnp.zeros((8, 128), jnp.int32)
        for s in range(NSEL):
            idx = jnp.argmax(val, axis=-1)[0].astype(jnp.int32)
            val = jnp.where(lanes == idx, -jnp.inf, val)
            cidx = jnp.minimum(idx, NBLK2 - 1)
            acc = jnp.where(s_of_g == s, (b * NBLK2 + cidx) * (BS * BS), acc)
        o_ref[0] = acc + jnp.where(g < NSEL * BS * BS, g % (BS * BS), 0)

    return pl.pallas_call(
        body,
        grid=(B,),
        in_specs=[pl.BlockSpec((1, 1, NBLK), lambda b: (b, 0, 0))],
        out_specs=pl.BlockSpec((1, 8, 128), lambda b: (b, 0, 0)),
        out_shape=jax.ShapeDtypeStruct((B, 8, 128), jnp.int32),
    )(imp)


def _sc_gather(idxf, kblk2, vblk2):
    per_w = _GROWS // _NWORK
    out_t = [jax.ShapeDtypeStruct((_GROWS, DIM), jnp.float32)] * 2
    mesh = plsc.VectorSubcoreMesh(core_axis_name="c", subcore_axis_name="s")

    @pl.kernel(out_type=out_t, mesh=mesh,
               scratch_types=[pltpu.VMEM((per_w,), jnp.int32),
                              pltpu.VMEM((per_w, DIM), jnp.float32),
                              pltpu.VMEM((per_w, DIM), jnp.float32),
                              pltpu.SemaphoreType.DMA,
                              pltpu.SemaphoreType.DMA])
    def sc_kernel(idx_ref, k_ref, v_ref, ko_ref, vo_ref, idx_v, rk, rv,
                  semk, semv):
        wid = jax.lax.axis_index("s") * 2 + jax.lax.axis_index("c")
        base = wid * per_w
        pltpu.sync_copy(idx_ref.at[pl.ds(base, per_w)], idx_v)
        ck = pltpu.async_copy(k_ref.at[idx_v], rk, semk)
        cv = pltpu.async_copy(v_ref.at[idx_v], rv, semv)
        ck.wait()
        pltpu.sync_copy(rk, ko_ref.at[pl.ds(base, per_w)])
        cv.wait()
        pltpu.sync_copy(rv, vo_ref.at[pl.ds(base, per_w)])

    return sc_kernel(idxf, kblk2, vblk2)


def _relayout_blocks(raw):
    def body(r_ref, o_ref):
        for s in range(NSEL):
            g = r_ref[0][s * BS * BS:(s + 1) * BS * BS]
            o_ref[0, pl.ds(s * DIM, DIM), :] = g.T

    return pl.pallas_call(
        body,
        grid=(B,),
        in_specs=[pl.BlockSpec((1, NSEL * BS * BS, DIM), lambda b: (b, 0, 0))],
        out_specs=pl.BlockSpec((1, NSEL * DIM, BS * BS), lambda b: (b, 0, 0)),
        out_shape=jax.ShapeDtypeStruct((B, NSEL * DIM, BS * BS), jnp.float32),
    )(raw)


def _blockify(x4):
    y = x4.reshape(B, _NB2, BS, _NB2, BS, DIM).transpose(0, 1, 3, 2, 4, 5)
    return y.reshape(B, NBLK2 * BS * BS, DIM)


_NW = 5
_WT = WS * WS
_WINB = 8
_NWIN = B * _NW * _NW


def _rel_pos_index_np(ws):
    coords = np.stack(np.meshgrid(np.arange(ws), np.arange(ws), indexing="ij"))
    cf = coords.reshape(2, -1)
    rel = (cf[:, :, None] - cf[:, None, :]).transpose(1, 2, 0).astype(np.int64).copy()
    rel[:, :, 0] += ws - 1
    rel[:, :, 1] += ws - 1
    rel[:, :, 0] *= 2 * ws - 1
    return rel.sum(-1)


_RPI = _rel_pos_index_np(WS)


_GT = _WINB * _WT


def _window_attn(xg, wqkv, bqkv, wpr, bpr, bias_big):
    def body(x_ref, wqkv_ref, bqkv_ref, bias_ref, wpr_ref, bpr_ref, o_ref):
        xx = x_ref[0]
        qkv = _dot(xx, wqkv_ref[...]) + bqkv_ref[...]
        outs = []
        for h in range(NUM_HEADS):
            qh = qkv[:, h * HD:(h + 1) * HD]
            kh = qkv[:, DIM + h * HD:DIM + (h + 1) * HD]
            vh = qkv[:, 2 * DIM + h * HD:2 * DIM + (h + 1) * HD]
            s = _dot_t(qh, kh) * SCALE + bias_ref[h]
            m = jnp.max(s, axis=-1, keepdims=True)
            e = jnp.exp(s - m)
            p = e * (1.0 / jnp.sum(e, axis=-1, keepdims=True))
            outs.append(_dot(p, vh))
        o = jnp.concatenate(outs, axis=-1)
        o_ref[0] = _dot(o, wpr_ref[...]) + bpr_ref[...]

    return pl.pallas_call(
        body,
        grid=(_NWIN // _WINB,),
        in_specs=[
            pl.BlockSpec((1, _GT, DIM), lambda i: (i, 0, 0)),
            pl.BlockSpec((DIM, 3 * DIM), lambda i: (0, 0)),
            pl.BlockSpec((1, 3 * DIM), lambda i: (0, 0)),
            pl.BlockSpec((NUM_HEADS, _GT, _GT), lambda i: (0, 0, 0)),
            pl.BlockSpec((DIM, DIM), lambda i: (0, 0)),
            pl.BlockSpec((1, DIM), lambda i: (0, 0)),
        ],
        out_specs=pl.BlockSpec((1, _GT, DIM), lambda i: (i, 0, 0)),
        out_shape=jax.ShapeDtypeStruct((_NWIN // _WINB, _GT, DIM),
                                       jnp.float32),
    )(xg, wqkv, bqkv.reshape(1, -1), bias_big, wpr, bpr.reshape(1, -1))


def _combine(x_seq, o_cmp, o_slc, o_win, wg, bg, wp, bp):
    M = x_seq.shape[0]
    TM = 1024

    def body(x_ref, c_ref, s_ref, w_ref, wg_ref, bg_ref, wp_ref, bp_ref, o_ref):
        g = jax.nn.sigmoid(_dot(x_ref[...], wg_ref[...]) + bg_ref[...])
        comb = (g[:, 0:1] * c_ref[...] + g[:, 1:2] * s_ref[...]
                + g[:, 2:3] * w_ref[...])
        o_ref[...] = _dot(comb, wp_ref[...]) + bp_ref[...]

    return pl.pallas_call(
        body,
        grid=(M // TM,),
        in_specs=[
            pl.BlockSpec((TM, DIM), lambda i: (i, 0)),
            pl.BlockSpec((TM, DIM), lambda i: (i, 0)),
            pl.BlockSpec((TM, DIM), lambda i: (i, 0)),
            pl.BlockSpec((TM, DIM), lambda i: (i, 0)),
            pl.BlockSpec((DIM, 3), lambda i: (0, 0)),
            pl.BlockSpec((1, 3), lambda i: (0, 0)),
            pl.BlockSpec((DIM, DIM), lambda i: (0, 0)),
            pl.BlockSpec((1, DIM), lambda i: (0, 0)),
        ],
        out_specs=pl.BlockSpec((TM, DIM), lambda i: (i, 0)),
        out_shape=jax.ShapeDtypeStruct((M, DIM), jnp.float32),
    )(x_seq, o_cmp, o_slc, o_win, wg, bg.reshape(1, 3), wp, bp.reshape(1, DIM))


def kernel(x, W_qkv, b_qkv, pos_embed, Wk1, bk1, Wk2, bk2, Wv1, bv1, Wv2, bv2,
           W_qkv_w, b_qkv_w, W_proj_w, b_proj_w, rel_bias, W_gate, b_gate,
           W_proj, b_proj):
    x_seq = x.reshape(B, DIM, N).transpose(0, 2, 1)
    xf = x_seq.reshape(B * N, DIM)

    qf, kf, vf = _qkv_proj(xf, W_qkv, b_qkv, 1024)
    q = qf.reshape(B, N, DIM)
    k4 = kf.reshape(B, H, W, DIM)
    v4 = vf.reshape(B, H, W, DIM)

    w1kr = Wk1.reshape(DIM, BS * BS, 2 * DIM).transpose(1, 0, 2).reshape(DIN, 2 * DIM)
    w1vr = Wv1.reshape(DIM, BS * BS, 2 * DIM).transpose(1, 0, 2).reshape(DIN, 2 * DIM)
    padd = pos_embed.reshape(DIN).reshape(DIM, BS * BS).T
    kcmp = _compress_half(_deinterleave(k4), padd, w1kr, bk1, Wk2, bk2)
    vcmp = _compress_half(_deinterleave(v4), jnp.zeros_like(padd), w1vr, bv1,
                          Wv2, bv2)

    out_cmp, imp = _attention(q, kcmp, vcmp, True)

    idxf = _select_rows(imp)[:, :2, :].reshape(_GROWS)
    kblk2 = _blockify(k4).reshape(B * NBLK2 * BS * BS, DIM)
    vblk2 = _blockify(v4).reshape(B * NBLK2 * BS * BS, DIM)
    kraw, vraw = _sc_gather(idxf, kblk2, vblk2)
    kslc = _relayout_blocks(kraw.reshape(B, NSEL * BS * BS, DIM)).reshape(
        B, NSEL * BS * BS, DIM)
    vslc = _relayout_blocks(vraw.reshape(B, NSEL * BS * BS, DIM)).reshape(
        B, NSEL * BS * BS, DIM)

    out_slc = _attention(q, kslc, vslc, False)

    xp = jnp.pad(x, ((0, 0), (0, 0), (0, _NW * WS - H), (0, _NW * WS - W)))
    xw = xp.reshape(B, DIM, _NW, WS, _NW, WS).transpose(0, 2, 4, 3, 5, 1)
    xg = xw.reshape(_NWIN // _WINB, _GT, DIM)
    bias = rel_bias[jnp.asarray(_RPI.reshape(-1))].reshape(_WT, _WT, NUM_HEADS)
    bias = bias.transpose(2, 0, 1)
    blockmask = jnp.kron(jnp.eye(_WINB, dtype=jnp.float32),
                         jnp.ones((_WT, _WT), jnp.float32)) > 0.5
    bias_big = jnp.where(blockmask[None], jnp.tile(bias, (1, _WINB, _WINB)),
                         -1e30)
    ow = _window_attn(xg, W_qkv_w, b_qkv_w, W_proj_w, b_proj_w, bias_big)
    ow = ow.reshape(B, _NW, _NW, WS, WS, DIM).transpose(0, 5, 1, 3, 2, 4)
    ow = ow.reshape(B, DIM, _NW * WS, _NW * WS)[:, :, :H, :W]
    out_win = ow.reshape(B, DIM, N).transpose(0, 2, 1).reshape(B * N, DIM)

    out = _combine(xf, out_cmp.reshape(B * N, DIM), out_slc.reshape(B * N, DIM),
                   out_win, W_gate, b_gate, W_proj, b_proj)
    return out.reshape(B, N, DIM).transpose(0, 2, 1).reshape(B, DIM, H, W)
